# baseline (speedup 1.0000x reference)
import jax
import jax.numpy as jnp
from jax.experimental import pallas as pl

N = 512
B = 64
L = 12
U = 64
K = 2
NUM_MAT = K + 1


def _bdot(a, b):
    return jnp.dot(a.astype(jnp.bfloat16), b,
                   preferred_element_type=jnp.float32)


def _layer_body(xin_ref, h_ref, s_ref, wg_ref, bg_ref, wc_ref, bc_ref, out_ref):
    s = s_ref[...]
    xin = xin_ref[0]
    h = h_ref[0]

    wg = wg_ref[...]
    bg = bg_ref[...]
    wc = wc_ref[...]
    bc = bc_ref[...]

    g0 = jnp.concatenate([xin, h], axis=1)
    g1 = jnp.dot(s, g0.astype(jnp.bfloat16), preferred_element_type=jnp.float32)
    g2 = 2.0 * jnp.dot(s, g1.astype(jnp.bfloat16), preferred_element_type=jnp.float32) - g0
    val = (_bdot(g0, wg[0]) + _bdot(g1, wg[1]) + _bdot(g2, wg[2]) + bg)
    val = jax.nn.sigmoid(val)
    r = val[:, :U]
    u = val[:, U:]

    c0 = jnp.concatenate([xin, r * h], axis=1)
    c1 = jnp.dot(s, c0.astype(jnp.bfloat16), preferred_element_type=jnp.float32)
    c2 = 2.0 * jnp.dot(s, c1.astype(jnp.bfloat16), preferred_element_type=jnp.float32) - c0
    c = jnp.tanh(_bdot(c0, wc[0]) + _bdot(c1, wc[1]) + _bdot(c2, wc[2]) + bc)

    out_ref[0] = u * h + (1.0 - u) * c


def _dcgru_layer(x_in, h, support, Wg, bg, Wc, bc):
    F = x_in.shape[-1]
    in_sz = F + U
    Wg3 = Wg.reshape(in_sz, NUM_MAT, 2 * U).transpose(1, 0, 2).astype(jnp.bfloat16)
    Wc3 = Wc.reshape(in_sz, NUM_MAT, U).transpose(1, 0, 2).astype(jnp.bfloat16)
    bg2 = bg.reshape(1, 2 * U)
    bc2 = bc.reshape(1, U)

    return pl.pallas_call(
        _layer_body,
        grid=(B,),
        in_specs=[
            pl.BlockSpec((1, N, F), lambda b: (b, 0, 0)),
            pl.BlockSpec((1, N, U), lambda b: (b, 0, 0)),
            pl.BlockSpec((N, N), lambda b: (0, 0)),
            pl.BlockSpec((NUM_MAT, in_sz, 2 * U), lambda b: (0, 0, 0)),
            pl.BlockSpec((1, 2 * U), lambda b: (0, 0)),
            pl.BlockSpec((NUM_MAT, in_sz, U), lambda b: (0, 0, 0)),
            pl.BlockSpec((1, U), lambda b: (0, 0)),
        ],
        out_specs=pl.BlockSpec((1, N, U), lambda b: (b, 0, 0)),
        out_shape=jax.ShapeDtypeStruct((B, N, U), jnp.float32),
    )(x_in, h, support, Wg3, bg2, Wc3, bc2)


@jax.jit
def kernel(inputs, hidden_state, support, Wg0, bg0, Wc0, bc0, Wg1, bg1, Wc1, bc1):
    x = inputs.reshape(B, N, L)
    h0_in = hidden_state[0].reshape(B, N, U)
    h1_in = hidden_state[1].reshape(B, N, U)
    s16 = support.astype(jnp.bfloat16)
    h0 = _dcgru_layer(x, h0_in, s16, Wg0, bg0, Wc0, bc0)
    h1 = _dcgru_layer(h0, h1_in, s16, Wg1, bg1, Wc1, bc1)
    h0f = h0.reshape(B, N * U)
    h1f = h1.reshape(B, N * U)
    return h1f, jnp.stack([h0f, h1f], axis=0)


# speedup vs baseline: 1.7369x; 1.7369x over previous
import jax
import jax.numpy as jnp
from jax.experimental import pallas as pl

N = 512
B = 64
L = 12
U = 64
K = 2
NUM_MAT = K + 1
BC = 4
SUB = 128


def _layer_body(xin_ref, h_ref, s_ref, wg_ref, bg_ref, wc_ref, bc_ref, out_ref):
    F = xin_ref.shape[-1]
    pad = SUB - (F + U)
    s = s_ref[...]
    wg = wg_ref[...]
    bg = bg_ref[...]
    wc = wc_ref[...]
    bc = bc_ref[...]

    xs = [xin_ref[i].astype(jnp.bfloat16) for i in range(BC)]
    hs = [h_ref[i] for i in range(BC)]
    zpad = jnp.zeros((N, pad), jnp.bfloat16) if pad else None

    def panel(states):
        parts = []
        for i in range(BC):
            parts.append(states[i])
            parts.append(xs[i])
            if pad:
                parts.append(zpad)
        return jnp.concatenate(parts, axis=1)

    def diffuse(p0):
        t1 = jnp.dot(s, p0, preferred_element_type=jnp.float32)
        p1 = t1.astype(jnp.bfloat16)
        p2 = jnp.dot(s, p1, preferred_element_type=jnp.float32).astype(jnp.bfloat16)
        return p1, p2

    def proj(p0, p1, p2, w, bias, i):
        sl = slice(i * SUB, (i + 1) * SUB)
        return (jnp.dot(p0[:, sl], w[0], preferred_element_type=jnp.float32)
                + jnp.dot(p1[:, sl], w[1], preferred_element_type=jnp.float32)
                + jnp.dot(p2[:, sl], w[2], preferred_element_type=jnp.float32)
                + bias)

    g0 = panel([h.astype(jnp.bfloat16) for h in hs])
    g1, g2 = diffuse(g0)
    rs, us = [], []
    for i in range(BC):
        val = jax.nn.sigmoid(proj(g0, g1, g2, wg, bg, i))
        rs.append(val[:, :U])
        us.append(val[:, U:])

    c0 = panel([(rs[i] * hs[i]).astype(jnp.bfloat16) for i in range(BC)])
    c1, c2 = diffuse(c0)
    for i in range(BC):
        c = jnp.tanh(proj(c0, c1, c2, wc, bc, i))
        out_ref[i] = us[i] * hs[i] + (1.0 - us[i]) * c


def _fold_weights(W, F, out):
    in_sz = F + U
    W3 = W.reshape(in_sz, NUM_MAT, out).transpose(1, 0, 2)
    W3 = jnp.stack([W3[0] - W3[2], W3[1], 2.0 * W3[2]], axis=0)
    W3 = jnp.concatenate([W3[:, F:, :], W3[:, :F, :],
                          jnp.zeros((NUM_MAT, SUB - in_sz, out), W3.dtype)],
                         axis=1)
    return W3.astype(jnp.bfloat16)


def _dcgru_layer(x_in, h, s16, Wg, bg, Wc, bc):
    F = x_in.shape[-1]
    Wg3 = _fold_weights(Wg, F, 2 * U)
    Wc3 = _fold_weights(Wc, F, U)
    bg2 = bg.reshape(1, 2 * U)
    bc2 = bc.reshape(1, U)

    return pl.pallas_call(
        _layer_body,
        grid=(B // BC,),
        in_specs=[
            pl.BlockSpec((BC, N, F), lambda b: (b, 0, 0)),
            pl.BlockSpec((BC, N, U), lambda b: (b, 0, 0)),
            pl.BlockSpec((N, N), lambda b: (0, 0)),
            pl.BlockSpec((NUM_MAT, SUB, 2 * U), lambda b: (0, 0, 0)),
            pl.BlockSpec((1, 2 * U), lambda b: (0, 0)),
            pl.BlockSpec((NUM_MAT, SUB, U), lambda b: (0, 0, 0)),
            pl.BlockSpec((1, U), lambda b: (0, 0)),
        ],
        out_specs=pl.BlockSpec((BC, N, U), lambda b: (b, 0, 0)),
        out_shape=jax.ShapeDtypeStruct((B, N, U), jnp.float32),
    )(x_in, h, s16, Wg3, bg2, Wc3, bc2)


@jax.jit
def kernel(inputs, hidden_state, support, Wg0, bg0, Wc0, bc0, Wg1, bg1, Wc1, bc1):
    x = inputs.reshape(B, N, L)
    h0_in = hidden_state[0].reshape(B, N, U)
    h1_in = hidden_state[1].reshape(B, N, U)
    s16 = support.astype(jnp.bfloat16)
    h0 = _dcgru_layer(x, h0_in, s16, Wg0, bg0, Wc0, bc0)
    h1 = _dcgru_layer(h0, h1_in, s16, Wg1, bg1, Wc1, bc1)
    h0f = h0.reshape(B, N * U)
    h1f = h1.reshape(B, N * U)
    return h1f, jnp.stack([h0f, h1f], axis=0)


# speedup vs baseline: 1.8373x; 1.0578x over previous
import jax
import jax.numpy as jnp
from jax.experimental import pallas as pl

N = 512
B = 64
L = 12
U = 64
K = 2
NUM_MAT = K + 1
BC = 8
SUB = 128


def _layer_body(xin_ref, h_ref, s_ref, wg_ref, bg_ref, wc_ref, bc_ref, out_ref):
    F = xin_ref.shape[-1]
    pad = SUB - (F + U)
    s = s_ref[...]
    wg = wg_ref[...]
    bg = bg_ref[...]
    wc = wc_ref[...]
    bc = bc_ref[...]

    xs = [xin_ref[i].astype(jnp.bfloat16) for i in range(BC)]
    hs = [h_ref[i] for i in range(BC)]
    zpad = jnp.zeros((N, pad), jnp.bfloat16) if pad else None

    def panel(states):
        parts = []
        for i in range(BC):
            parts.append(states[i])
            parts.append(xs[i])
            if pad:
                parts.append(zpad)
        return jnp.concatenate(parts, axis=1)

    def diffuse(p0):
        t1 = jnp.dot(s, p0, preferred_element_type=jnp.float32)
        p1 = t1.astype(jnp.bfloat16)
        p2 = jnp.dot(s, p1, preferred_element_type=jnp.float32).astype(jnp.bfloat16)
        return p1, p2

    def proj(p0, p1, p2, w, bias, i):
        sl = slice(i * SUB, (i + 1) * SUB)
        return (jnp.dot(p0[:, sl], w[0], preferred_element_type=jnp.float32)
                + jnp.dot(p1[:, sl], w[1], preferred_element_type=jnp.float32)
                + jnp.dot(p2[:, sl], w[2], preferred_element_type=jnp.float32)
                + bias)

    g0 = panel([h.astype(jnp.bfloat16) for h in hs])
    g1, g2 = diffuse(g0)
    rs, us = [], []
    for i in range(BC):
        val = jax.nn.sigmoid(proj(g0, g1, g2, wg, bg, i))
        rs.append(val[:, :U])
        us.append(val[:, U:])

    c0 = panel([(rs[i] * hs[i]).astype(jnp.bfloat16) for i in range(BC)])
    c1, c2 = diffuse(c0)
    for i in range(BC):
        c = jnp.tanh(proj(c0, c1, c2, wc, bc, i))
        out_ref[i] = us[i] * hs[i] + (1.0 - us[i]) * c


def _fold_weights(W, F, out):
    in_sz = F + U
    W3 = W.reshape(in_sz, NUM_MAT, out).transpose(1, 0, 2)
    W3 = jnp.stack([W3[0] - W3[2], W3[1], 2.0 * W3[2]], axis=0)
    W3 = jnp.concatenate([W3[:, F:, :], W3[:, :F, :],
                          jnp.zeros((NUM_MAT, SUB - in_sz, out), W3.dtype)],
                         axis=1)
    return W3.astype(jnp.bfloat16)


def _dcgru_layer(x_in, h, s16, Wg, bg, Wc, bc):
    F = x_in.shape[-1]
    Wg3 = _fold_weights(Wg, F, 2 * U)
    Wc3 = _fold_weights(Wc, F, U)
    bg2 = bg.reshape(1, 2 * U)
    bc2 = bc.reshape(1, U)

    return pl.pallas_call(
        _layer_body,
        grid=(B // BC,),
        in_specs=[
            pl.BlockSpec((BC, N, F), lambda b: (b, 0, 0)),
            pl.BlockSpec((BC, N, U), lambda b: (b, 0, 0)),
            pl.BlockSpec((N, N), lambda b: (0, 0)),
            pl.BlockSpec((NUM_MAT, SUB, 2 * U), lambda b: (0, 0, 0)),
            pl.BlockSpec((1, 2 * U), lambda b: (0, 0)),
            pl.BlockSpec((NUM_MAT, SUB, U), lambda b: (0, 0, 0)),
            pl.BlockSpec((1, U), lambda b: (0, 0)),
        ],
        out_specs=pl.BlockSpec((BC, N, U), lambda b: (b, 0, 0)),
        out_shape=jax.ShapeDtypeStruct((B, N, U), jnp.float32),
    )(x_in, h, s16, Wg3, bg2, Wc3, bc2)


@jax.jit
def kernel(inputs, hidden_state, support, Wg0, bg0, Wc0, bc0, Wg1, bg1, Wc1, bc1):
    x = inputs.reshape(B, N, L)
    h0_in = hidden_state[0].reshape(B, N, U)
    h1_in = hidden_state[1].reshape(B, N, U)
    s16 = support.astype(jnp.bfloat16)
    h0 = _dcgru_layer(x, h0_in, s16, Wg0, bg0, Wc0, bc0)
    h1 = _dcgru_layer(h0, h1_in, s16, Wg1, bg1, Wc1, bc1)
    h0f = h0.reshape(B, N * U)
    h1f = h1.reshape(B, N * U)
    return h1f, jnp.stack([h0f, h1f], axis=0)


# speedup vs baseline: 1.9960x; 1.0864x over previous
import jax
import jax.numpy as jnp
from jax.experimental import pallas as pl

N = 512
B = 64
L = 12
U = 64
K = 2
NUM_MAT = K + 1
BC = 8
SUB = 128


def _dcgru_chunk(xs, hs, s, wg, bg, wc, bc, pad):
    zpad = jnp.zeros((N, pad), jnp.bfloat16) if pad else None

    def panel(states):
        parts = []
        for i in range(BC):
            parts.append(states[i])
            parts.append(xs[i])
            if pad:
                parts.append(zpad)
        return jnp.concatenate(parts, axis=1)

    def diffuse(p0):
        p1 = jnp.dot(s, p0, preferred_element_type=jnp.float32).astype(jnp.bfloat16)
        p2 = jnp.dot(s, p1, preferred_element_type=jnp.float32).astype(jnp.bfloat16)
        return p1, p2

    def proj(p0, p1, p2, w, bias, i):
        sl = slice(i * SUB, (i + 1) * SUB)
        return (jnp.dot(p0[:, sl], w[0], preferred_element_type=jnp.float32)
                + jnp.dot(p1[:, sl], w[1], preferred_element_type=jnp.float32)
                + jnp.dot(p2[:, sl], w[2], preferred_element_type=jnp.float32)
                + bias)

    g0 = panel([h.astype(jnp.bfloat16) for h in hs])
    g1, g2 = diffuse(g0)
    rs, us = [], []
    for i in range(BC):
        val = jax.nn.sigmoid(proj(g0, g1, g2, wg, bg, i))
        rs.append(val[:, :U])
        us.append(val[:, U:])

    c0 = panel([(rs[i] * hs[i]).astype(jnp.bfloat16) for i in range(BC)])
    c1, c2 = diffuse(c0)
    outs = []
    for i in range(BC):
        c = jnp.tanh(proj(c0, c1, c2, wc, bc, i))
        outs.append(us[i] * hs[i] + (1.0 - us[i]) * c)
    return outs


def _body(x_ref, h0_ref, h1_ref, s_ref,
          wg0_ref, bg0_ref, wc0_ref, bc0_ref,
          wg1_ref, bg1_ref, wc1_ref, bc1_ref,
          hid_ref, out_ref):
    s = s_ref[...]

    xs0 = [x_ref[i].astype(jnp.bfloat16) for i in range(BC)]
    hs0 = [h0_ref[i] for i in range(BC)]
    h0n = _dcgru_chunk(xs0, hs0, s, wg0_ref[...], bg0_ref[...],
                       wc0_ref[...], bc0_ref[...], SUB - (L + U))

    xs1 = [h.astype(jnp.bfloat16) for h in h0n]
    hs1 = [h1_ref[i] for i in range(BC)]
    h1n = _dcgru_chunk(xs1, hs1, s, wg1_ref[...], bg1_ref[...],
                       wc1_ref[...], bc1_ref[...], SUB - (U + U))

    for i in range(BC):
        hid_ref[0, i] = h0n[i]
        hid_ref[1, i] = h1n[i]
        out_ref[i] = h1n[i]


def _fold_weights(W, F, out):
    in_sz = F + U
    W3 = W.reshape(in_sz, NUM_MAT, out).transpose(1, 0, 2)
    W3 = jnp.stack([W3[0] - W3[2], W3[1], 2.0 * W3[2]], axis=0)
    W3 = jnp.concatenate([W3[:, F:, :], W3[:, :F, :],
                          jnp.zeros((NUM_MAT, SUB - in_sz, out), W3.dtype)],
                         axis=1)
    return W3.astype(jnp.bfloat16)


@jax.jit
def kernel(inputs, hidden_state, support, Wg0, bg0, Wc0, bc0, Wg1, bg1, Wc1, bc1):
    x = inputs.reshape(B, N, L)
    h0_in = hidden_state[0].reshape(B, N, U)
    h1_in = hidden_state[1].reshape(B, N, U)
    s16 = support.astype(jnp.bfloat16)
    args = (x, h0_in, h1_in, s16,
            _fold_weights(Wg0, L, 2 * U), bg0.reshape(1, 2 * U),
            _fold_weights(Wc0, L, U), bc0.reshape(1, U),
            _fold_weights(Wg1, U, 2 * U), bg1.reshape(1, 2 * U),
            _fold_weights(Wc1, U, U), bc1.reshape(1, U))

    const = lambda b: (0, 0)
    const3 = lambda b: (0, 0, 0)
    hid, out = pl.pallas_call(
        _body,
        grid=(B // BC,),
        in_specs=[
            pl.BlockSpec((BC, N, L), lambda b: (b, 0, 0)),
            pl.BlockSpec((BC, N, U), lambda b: (b, 0, 0)),
            pl.BlockSpec((BC, N, U), lambda b: (b, 0, 0)),
            pl.BlockSpec((N, N), const),
            pl.BlockSpec((NUM_MAT, SUB, 2 * U), const3),
            pl.BlockSpec((1, 2 * U), const),
            pl.BlockSpec((NUM_MAT, SUB, U), const3),
            pl.BlockSpec((1, U), const),
            pl.BlockSpec((NUM_MAT, SUB, 2 * U), const3),
            pl.BlockSpec((1, 2 * U), const),
            pl.BlockSpec((NUM_MAT, SUB, U), const3),
            pl.BlockSpec((1, U), const),
        ],
        out_specs=[
            pl.BlockSpec((2, BC, N, U), lambda b: (0, b, 0, 0)),
            pl.BlockSpec((BC, N, U), lambda b: (b, 0, 0)),
        ],
        out_shape=[
            jax.ShapeDtypeStruct((2, B, N, U), jnp.float32),
            jax.ShapeDtypeStruct((B, N, U), jnp.float32),
        ],
    )(*args)
    return out.reshape(B, N * U), hid.reshape(2, B, N * U)


# speedup vs baseline: 2.0470x; 1.0255x over previous
import jax
import jax.numpy as jnp
from jax.experimental import pallas as pl

N = 512
B = 64
L = 12
U = 64
K = 2
NUM_MAT = K + 1
BC = 16
SUB = 128


def _dcgru_chunk(xs, hs, s, wg, bg, wc, bc, pad):
    zpad = jnp.zeros((N, pad), jnp.bfloat16) if pad else None

    def panel(states):
        parts = []
        for i in range(BC):
            parts.append(states[i])
            parts.append(xs[i])
            if pad:
                parts.append(zpad)
        return jnp.concatenate(parts, axis=1)

    def diffuse(p0):
        p1 = jnp.dot(s, p0, preferred_element_type=jnp.float32).astype(jnp.bfloat16)
        p2 = jnp.dot(s, p1, preferred_element_type=jnp.float32).astype(jnp.bfloat16)
        return p1, p2

    def proj(p0, p1, p2, w, bias, i):
        sl = slice(i * SUB, (i + 1) * SUB)
        return (jnp.dot(p0[:, sl], w[0], preferred_element_type=jnp.float32)
                + jnp.dot(p1[:, sl], w[1], preferred_element_type=jnp.float32)
                + jnp.dot(p2[:, sl], w[2], preferred_element_type=jnp.float32)
                + bias)

    g0 = panel([h.astype(jnp.bfloat16) for h in hs])
    g1, g2 = diffuse(g0)
    rs, us = [], []
    for i in range(BC):
        val = jax.nn.sigmoid(proj(g0, g1, g2, wg, bg, i))
        rs.append(val[:, :U])
        us.append(val[:, U:])

    c0 = panel([(rs[i] * hs[i]).astype(jnp.bfloat16) for i in range(BC)])
    c1, c2 = diffuse(c0)
    outs = []
    for i in range(BC):
        c = jnp.tanh(proj(c0, c1, c2, wc, bc, i))
        outs.append(us[i] * hs[i] + (1.0 - us[i]) * c)
    return outs


def _body(x_ref, h0_ref, h1_ref, s_ref,
          wg0_ref, bg0_ref, wc0_ref, bc0_ref,
          wg1_ref, bg1_ref, wc1_ref, bc1_ref,
          hid_ref):
    s = s_ref[...]

    xs0 = [x_ref[i].astype(jnp.bfloat16) for i in range(BC)]
    hs0 = [h0_ref[i] for i in range(BC)]
    h0n = _dcgru_chunk(xs0, hs0, s, wg0_ref[...], bg0_ref[...],
                       wc0_ref[...], bc0_ref[...], SUB - (L + U))

    xs1 = [h.astype(jnp.bfloat16) for h in h0n]
    hs1 = [h1_ref[i] for i in range(BC)]
    h1n = _dcgru_chunk(xs1, hs1, s, wg1_ref[...], bg1_ref[...],
                       wc1_ref[...], bc1_ref[...], SUB - (U + U))

    for i in range(BC):
        hid_ref[0, i] = h0n[i]
        hid_ref[1, i] = h1n[i]


def _fold_weights(W, F, out):
    in_sz = F + U
    W3 = W.reshape(in_sz, NUM_MAT, out).transpose(1, 0, 2)
    W3 = jnp.stack([W3[0] - W3[2], W3[1], 2.0 * W3[2]], axis=0)
    W3 = jnp.concatenate([W3[:, F:, :], W3[:, :F, :],
                          jnp.zeros((NUM_MAT, SUB - in_sz, out), W3.dtype)],
                         axis=1)
    return W3.astype(jnp.bfloat16)


@jax.jit
def kernel(inputs, hidden_state, support, Wg0, bg0, Wc0, bc0, Wg1, bg1, Wc1, bc1):
    x = inputs.reshape(B, N, L)
    h0_in = hidden_state[0].reshape(B, N, U)
    h1_in = hidden_state[1].reshape(B, N, U)
    s16 = support.astype(jnp.bfloat16)
    args = (x, h0_in, h1_in, s16,
            _fold_weights(Wg0, L, 2 * U), bg0.reshape(1, 2 * U),
            _fold_weights(Wc0, L, U), bc0.reshape(1, U),
            _fold_weights(Wg1, U, 2 * U), bg1.reshape(1, 2 * U),
            _fold_weights(Wc1, U, U), bc1.reshape(1, U))

    const = lambda b: (0, 0)
    const3 = lambda b: (0, 0, 0)
    hid = pl.pallas_call(
        _body,
        grid=(B // BC,),
        in_specs=[
            pl.BlockSpec((BC, N, L), lambda b: (b, 0, 0)),
            pl.BlockSpec((BC, N, U), lambda b: (b, 0, 0)),
            pl.BlockSpec((BC, N, U), lambda b: (b, 0, 0)),
            pl.BlockSpec((N, N), const),
            pl.BlockSpec((NUM_MAT, SUB, 2 * U), const3),
            pl.BlockSpec((1, 2 * U), const),
            pl.BlockSpec((NUM_MAT, SUB, U), const3),
            pl.BlockSpec((1, U), const),
            pl.BlockSpec((NUM_MAT, SUB, 2 * U), const3),
            pl.BlockSpec((1, 2 * U), const),
            pl.BlockSpec((NUM_MAT, SUB, U), const3),
            pl.BlockSpec((1, U), const),
        ],
        out_specs=pl.BlockSpec((2, BC, N, U), lambda b: (0, b, 0, 0)),
        out_shape=jax.ShapeDtypeStruct((2, B, N, U), jnp.float32),
    )(*args)
    hid = hid.reshape(2, B, N * U)
    return hid[1], hid
